# baseline (speedup 1.0000x reference)
import functools

import jax
import jax.numpy as jnp
from jax import lax
from jax.experimental import pallas as pl
from jax.experimental.pallas import tpu as pltpu
from jax.experimental.pallas import tpu_sc as plsc

B, N, D, H, L, TOPK = 16, 4096, 768, 12, 2, 16
HID = D // H
DH = D // H
S = TOPK + 1
SCORE_ROWS = 1024

_HI = lax.Precision.HIGHEST


def _dot_t(a, w):
    return lax.dot_general(a, w, (((1,), (1,)), ((), ())),
                           precision=_HI, preferred_element_type=jnp.float32)


def _ln(x, g, b):
    m = jnp.mean(x, axis=-1, keepdims=True)
    xc = x - m
    v = jnp.mean(xc * xc, axis=-1, keepdims=True)
    return xc / jnp.sqrt(v + 1e-5) * g + b


def _score_body(x_ref, g_ref, b_ref, w1_ref, b1_ref, w2_ref, b2_ref, out_ref):
    xb = x_ref[...]
    ln = _ln(xb, g_ref[...], b_ref[...])
    h = lax.dot_general(ln.astype(jnp.bfloat16), w1_ref[...].astype(jnp.bfloat16),
                        (((1,), (1,)), ((), ())),
                        preferred_element_type=jnp.float32) + b1_ref[...]
    h = h * 0.5 * (1.0 + lax.erf(h * (1.0 / jnp.sqrt(2.0)).astype(jnp.float32)))
    hb = h.astype(jnp.bfloat16).astype(jnp.float32)
    wb = w2_ref[...].astype(jnp.bfloat16).astype(jnp.float32)
    s = jnp.sum(hb * wb, axis=1, keepdims=True) + b2_ref[...]
    out_ref[...] = jax.nn.sigmoid(s)


def _score(x2d, g, b, w1, b1, w2row, b2):
    bn = x2d.shape[0]
    return pl.pallas_call(
        _score_body,
        grid=(bn // SCORE_ROWS,),
        in_specs=[
            pl.BlockSpec((SCORE_ROWS, D), lambda i: (i, 0)),
            pl.BlockSpec((D,), lambda i: (0,)),
            pl.BlockSpec((D,), lambda i: (0,)),
            pl.BlockSpec((HID, D), lambda i: (0, 0)),
            pl.BlockSpec((HID,), lambda i: (0,)),
            pl.BlockSpec((HID,), lambda i: (0,)),
            pl.BlockSpec((1,), lambda i: (0,)),
        ],
        out_specs=pl.BlockSpec((SCORE_ROWS, 1), lambda i: (i, 0)),
        out_shape=jax.ShapeDtypeStruct((bn, 1), jnp.float32),
    )(x2d, g, b, w1, b1, w2row, b2)


def _topk_gather(w, xflat):
    mesh = plsc.VectorSubcoreMesh(core_axis_name="c", subcore_axis_name="s")

    @functools.partial(
        pl.kernel,
        mesh=mesh,
        compiler_params=pltpu.CompilerParams(needs_layout_passes=False),
        out_type=(jax.ShapeDtypeStruct((B * TOPK, D), jnp.float32),
                  jax.ShapeDtypeStruct((B * TOPK,), jnp.float32)),
        scratch_types=[
            pltpu.VMEM((N,), jnp.float32),
            pltpu.VMEM((TOPK,), jnp.int32),
            pltpu.VMEM((TOPK,), jnp.float32),
            pltpu.VMEM((TOPK, D), jnp.float32),
            pltpu.SemaphoreType.DMA,
        ],
    )
    def k(w_hbm, x_hbm, rows_out, vals_out, w_v, idx_v, val_v, rows_v, sem):
        wid = lax.axis_index("s") * 2 + lax.axis_index("c")

        @pl.when(wid < B)
        def _():
            bag = wid
            pltpu.sync_copy(w_hbm.at[bag], w_v)
            lanes = lax.iota(jnp.int32, 16)
            cur_v, cur_i = plsc.sort_key_val(w_v[pl.ds(0, 16)], lanes)

            def body(c, carry):
                cv, ci = carry
                sv, si = plsc.sort_key_val(w_v[pl.ds(c * 16, 16)],
                                           lanes + c * 16)
                rv = lax.rev(sv, (0,))
                ri = lax.rev(si, (0,))
                keep = cv >= rv
                nv, ni = plsc.sort_key_val(jnp.where(keep, cv, rv),
                                           jnp.where(keep, ci, ri))
                return (nv, ni)

            cur_v, cur_i = lax.fori_loop(1, N // 16, body, (cur_v, cur_i))
            idx_v[...] = cur_i + bag * N
            val_v[...] = cur_v
            pltpu.async_copy(x_hbm.at[idx_v], rows_v, sem).wait()
            pltpu.sync_copy(rows_v, rows_out.at[pl.ds(bag * TOPK, TOPK)])
            pltpu.sync_copy(val_v, vals_out.at[pl.ds(bag * TOPK, TOPK)])

    return k(w, xflat)


_T = B * S


def _bdot(a, w):
    return lax.dot_general(a.astype(jnp.bfloat16), w.astype(jnp.bfloat16),
                           (((1,), (1,)), ((), ())),
                           preferred_element_type=jnp.float32)


def _attn_body(emb_ref, tw_ref, cls_ref, g_ref, b_ref, inw_ref, inb_ref,
               outw_ref, outb_ref, o_ref, z_scr, a_scr):
    g = g_ref[...]
    bb = b_ref[...]
    scale = (1.0 / jnp.sqrt(float(DH))).astype(jnp.float32)

    def layer(a, cls_only):
        qkv = _bdot(a, inw_ref[0]) + inb_ref[0]
        q = qkv[:, :D]
        kk = qkv[:, D:2 * D]
        vv = qkv[:, 2 * D:]
        if cls_only:
            q = q.reshape(B, S, D)[:, 0, :]
            m = lax.broadcasted_iota(jnp.int32, (B, _T), 0) == \
                lax.broadcasted_iota(jnp.int32, (B, _T), 1) // S
        else:
            m = lax.broadcasted_iota(jnp.int32, (_T, _T), 0) // S == \
                lax.broadcasted_iota(jnp.int32, (_T, _T), 1) // S
        outs = []
        for hh in range(H):
            lo = hh * DH
            sc = _bdot(q[:, lo:lo + DH], kk[:, lo:lo + DH]) * scale
            sc = jnp.where(m, sc, -1e30)
            sc = sc - jnp.max(sc, axis=1, keepdims=True)
            e = jnp.exp(sc)
            p = e / jnp.sum(e, axis=1, keepdims=True)
            outs.append(lax.dot_general(
                p.astype(jnp.bfloat16), vv[:, lo:lo + DH].astype(jnp.bfloat16),
                (((1,), (0,)), ((), ())),
                preferred_element_type=jnp.float32))
        o = jnp.concatenate(outs, axis=1)
        return _bdot(o, outw_ref[0]) + outb_ref[0]

    @pl.when(pl.program_id(0) == 0)
    def _():
        ew = (emb_ref[...] * tw_ref[...]).reshape(B, TOPK, D)
        cls = jnp.broadcast_to(cls_ref[...].reshape(1, 1, D), (B, 1, D))
        zc = jnp.concatenate([cls, ew], axis=1).reshape(_T, D)
        z = _ln(zc, g, bb)
        z_scr[...] = z
        a_scr[...] = _ln(z + layer(z, False), g, bb)

    @pl.when(pl.program_id(0) == 1)
    def _():
        o2 = layer(a_scr[...], True)
        z_cls = z_scr[...].reshape(B, S, D)[:, 0, :]
        o_ref[...] = _ln(z_cls + o2, g, bb)


def _attn(emb, tw, cls, g, b, in_w, in_b, out_w, out_b):
    return pl.pallas_call(
        _attn_body,
        grid=(L,),
        in_specs=[
            pl.BlockSpec((B * TOPK, D), lambda l: (0, 0)),
            pl.BlockSpec((B * TOPK, 1), lambda l: (0, 0)),
            pl.BlockSpec((1, D), lambda l: (0, 0)),
            pl.BlockSpec((D,), lambda l: (0,)),
            pl.BlockSpec((D,), lambda l: (0,)),
            pl.BlockSpec((1, 3 * D, D), lambda l: (l, 0, 0)),
            pl.BlockSpec((1, 1, 3 * D), lambda l: (l, 0, 0)),
            pl.BlockSpec((1, D, D), lambda l: (l, 0, 0)),
            pl.BlockSpec((1, 1, D), lambda l: (l, 0, 0)),
        ],
        out_specs=pl.BlockSpec((B, D), lambda l: (0, 0)),
        out_shape=jax.ShapeDtypeStruct((B, D), jnp.float32),
        scratch_shapes=[pltpu.VMEM((_T, D), jnp.float32),
                        pltpu.VMEM((_T, D), jnp.float32)],
    )(emb, tw, cls, g, b, in_w, in_b.reshape(L, 1, 3 * D), out_w,
      out_b.reshape(L, 1, D))


def kernel(x, cls_token, ln_g, ln_b, aw_ln_g, aw_ln_b, aw_w1, aw_b1, aw_w2,
           aw_b2, in_w, in_b, out_w, out_b):
    x2d = x.reshape(B * N, D)
    w = _score(x2d, aw_ln_g, aw_ln_b, aw_w1, aw_b1, aw_w2.reshape(HID), aw_b2)
    rows, vals = _topk_gather(w.reshape(B, N), x2d)
    return _attn(rows, vals.reshape(B * TOPK, 1),
                 cls_token.reshape(1, D), ln_g, ln_b, in_w, in_b, out_w, out_b)


# speedup vs baseline: 1.1309x; 1.1309x over previous
import functools

import jax
import jax.numpy as jnp
from jax import lax
from jax.experimental import pallas as pl
from jax.experimental.pallas import tpu as pltpu
from jax.experimental.pallas import tpu_sc as plsc

B, N, D, H, L, TOPK = 16, 4096, 768, 12, 2, 16
HID = D // H
DH = D // H
S = TOPK + 1
SCORE_ROWS = 2048

_HI = lax.Precision.HIGHEST


def _dot_t(a, w):
    return lax.dot_general(a, w, (((1,), (1,)), ((), ())),
                           precision=_HI, preferred_element_type=jnp.float32)


def _ln(x, g, b):
    m = jnp.mean(x, axis=-1, keepdims=True)
    xc = x - m
    v = jnp.mean(xc * xc, axis=-1, keepdims=True)
    return xc / jnp.sqrt(v + 1e-5) * g + b


def _score_body(x_ref, g_ref, b_ref, w1_ref, b1_ref, w2_ref, b2_ref, out_ref):
    xb = x_ref[...]
    ln = _ln(xb, g_ref[...], b_ref[...])
    h = lax.dot_general(ln.astype(jnp.bfloat16), w1_ref[...].astype(jnp.bfloat16),
                        (((1,), (1,)), ((), ())),
                        preferred_element_type=jnp.float32) + b1_ref[...]
    h = h * 0.5 * (1.0 + lax.erf(h * (1.0 / jnp.sqrt(2.0)).astype(jnp.float32)))
    hb = h.astype(jnp.bfloat16).astype(jnp.float32)
    wb = w2_ref[...].astype(jnp.bfloat16).astype(jnp.float32)
    s = jnp.sum(hb * wb, axis=1, keepdims=True) + b2_ref[...]
    out_ref[...] = jax.nn.sigmoid(s)


def _score(x2d, g, b, w1, b1, w2row, b2):
    bn = x2d.shape[0]
    return pl.pallas_call(
        _score_body,
        grid=(bn // SCORE_ROWS,),
        in_specs=[
            pl.BlockSpec((SCORE_ROWS, D), lambda i: (i, 0)),
            pl.BlockSpec((D,), lambda i: (0,)),
            pl.BlockSpec((D,), lambda i: (0,)),
            pl.BlockSpec((HID, D), lambda i: (0, 0)),
            pl.BlockSpec((HID,), lambda i: (0,)),
            pl.BlockSpec((HID,), lambda i: (0,)),
            pl.BlockSpec((1,), lambda i: (0,)),
        ],
        out_specs=pl.BlockSpec((SCORE_ROWS, 1), lambda i: (i, 0)),
        out_shape=jax.ShapeDtypeStruct((bn, 1), jnp.float32),
    )(x2d, g, b, w1, b1, w2row, b2)


def _topk_gather(w, xflat):
    mesh = plsc.VectorSubcoreMesh(core_axis_name="c", subcore_axis_name="s")

    @functools.partial(
        pl.kernel,
        mesh=mesh,
        compiler_params=pltpu.CompilerParams(needs_layout_passes=False),
        out_type=(jax.ShapeDtypeStruct((B * TOPK, D), jnp.float32),
                  jax.ShapeDtypeStruct((B * TOPK,), jnp.float32)),
        scratch_types=[
            pltpu.VMEM((N,), jnp.float32),
            pltpu.VMEM((TOPK,), jnp.int32),
            pltpu.VMEM((TOPK,), jnp.float32),
            pltpu.VMEM((TOPK, D), jnp.float32),
            pltpu.SemaphoreType.DMA,
        ],
    )
    def k(w_hbm, x_hbm, rows_out, vals_out, w_v, idx_v, val_v, rows_v, sem):
        wid = lax.axis_index("s") * 2 + lax.axis_index("c")

        @pl.when(wid < B)
        def _():
            bag = wid
            pltpu.sync_copy(w_hbm.at[bag], w_v)
            lanes = lax.iota(jnp.int32, 16)
            cur_v, cur_i = plsc.sort_key_val(w_v[pl.ds(0, 16)], lanes)

            def body(c, carry):
                cv, ci = carry
                sv, si = plsc.sort_key_val(w_v[pl.ds(c * 16, 16)],
                                           lanes + c * 16)
                rv = lax.rev(sv, (0,))
                ri = lax.rev(si, (0,))
                keep = cv >= rv
                nv, ni = plsc.sort_key_val(jnp.where(keep, cv, rv),
                                           jnp.where(keep, ci, ri))
                return (nv, ni)

            cur_v, cur_i = lax.fori_loop(1, N // 16, body, (cur_v, cur_i))
            idx_v[...] = cur_i + bag * N
            val_v[...] = cur_v
            pltpu.async_copy(x_hbm.at[idx_v], rows_v, sem).wait()
            pltpu.sync_copy(rows_v, rows_out.at[pl.ds(bag * TOPK, TOPK)])
            pltpu.sync_copy(val_v, vals_out.at[pl.ds(bag * TOPK, TOPK)])

    return k(w, xflat)


_T = B * S


def _bdot(a, w):
    return lax.dot_general(a.astype(jnp.bfloat16), w.astype(jnp.bfloat16),
                           (((1,), (1,)), ((), ())),
                           preferred_element_type=jnp.float32)


def _attn_body(emb_ref, tw_ref, cls_ref, g_ref, b_ref, inw_ref, inb_ref,
               outw_ref, outb_ref, o_ref, z_scr, a_scr):
    g = g_ref[...]
    bb = b_ref[...]
    scale = (1.0 / jnp.sqrt(float(DH))).astype(jnp.float32)

    def layer(a, cls_only):
        qkv = _bdot(a, inw_ref[0]) + inb_ref[0]
        q = qkv[:, :D]
        kk = qkv[:, D:2 * D]
        vv = qkv[:, 2 * D:]
        if cls_only:
            q = q.reshape(B, S, D)[:, 0, :]
            m = lax.broadcasted_iota(jnp.int32, (B, _T), 0) == \
                lax.broadcasted_iota(jnp.int32, (B, _T), 1) // S
        else:
            m = lax.broadcasted_iota(jnp.int32, (_T, _T), 0) // S == \
                lax.broadcasted_iota(jnp.int32, (_T, _T), 1) // S
        outs = []
        for hh in range(H):
            lo = hh * DH
            sc = _bdot(q[:, lo:lo + DH], kk[:, lo:lo + DH]) * scale
            sc = jnp.where(m, sc, -1e30)
            sc = sc - jnp.max(sc, axis=1, keepdims=True)
            e = jnp.exp(sc)
            p = e / jnp.sum(e, axis=1, keepdims=True)
            outs.append(lax.dot_general(
                p.astype(jnp.bfloat16), vv[:, lo:lo + DH].astype(jnp.bfloat16),
                (((1,), (0,)), ((), ())),
                preferred_element_type=jnp.float32))
        o = jnp.concatenate(outs, axis=1)
        return _bdot(o, outw_ref[0]) + outb_ref[0]

    @pl.when(pl.program_id(0) == 0)
    def _():
        ew = (emb_ref[...] * tw_ref[...]).reshape(B, TOPK, D)
        cls = jnp.broadcast_to(cls_ref[...].reshape(1, 1, D), (B, 1, D))
        zc = jnp.concatenate([cls, ew], axis=1).reshape(_T, D)
        z = _ln(zc, g, bb)
        z_scr[...] = z
        a_scr[...] = _ln(z + layer(z, False), g, bb)

    @pl.when(pl.program_id(0) == 1)
    def _():
        o2 = layer(a_scr[...], True)
        z_cls = z_scr[...].reshape(B, S, D)[:, 0, :]
        o_ref[...] = _ln(z_cls + o2, g, bb)


def _attn(emb, tw, cls, g, b, in_w, in_b, out_w, out_b):
    return pl.pallas_call(
        _attn_body,
        grid=(L,),
        in_specs=[
            pl.BlockSpec((B * TOPK, D), lambda l: (0, 0)),
            pl.BlockSpec((B * TOPK, 1), lambda l: (0, 0)),
            pl.BlockSpec((1, D), lambda l: (0, 0)),
            pl.BlockSpec((D,), lambda l: (0,)),
            pl.BlockSpec((D,), lambda l: (0,)),
            pl.BlockSpec((1, 3 * D, D), lambda l: (l, 0, 0)),
            pl.BlockSpec((1, 1, 3 * D), lambda l: (l, 0, 0)),
            pl.BlockSpec((1, D, D), lambda l: (l, 0, 0)),
            pl.BlockSpec((1, 1, D), lambda l: (l, 0, 0)),
        ],
        out_specs=pl.BlockSpec((B, D), lambda l: (0, 0)),
        out_shape=jax.ShapeDtypeStruct((B, D), jnp.float32),
        scratch_shapes=[pltpu.VMEM((_T, D), jnp.float32),
                        pltpu.VMEM((_T, D), jnp.float32)],
    )(emb, tw, cls, g, b, in_w, in_b.reshape(L, 1, 3 * D), out_w,
      out_b.reshape(L, 1, D))


def kernel(x, cls_token, ln_g, ln_b, aw_ln_g, aw_ln_b, aw_w1, aw_b1, aw_w2,
           aw_b2, in_w, in_b, out_w, out_b):
    x2d = x.reshape(B * N, D)
    w = _score(x2d, aw_ln_g, aw_ln_b, aw_w1, aw_b1, aw_w2.reshape(HID), aw_b2)
    rows, vals = _topk_gather(w.reshape(B, N), x2d)
    return _attn(rows, vals.reshape(B * TOPK, 1),
                 cls_token.reshape(1, D), ln_g, ln_b, in_w, in_b, out_w, out_b)


# speedup vs baseline: 1.1923x; 1.0544x over previous
import functools

import jax
import jax.numpy as jnp
from jax import lax
from jax.experimental import pallas as pl
from jax.experimental.pallas import tpu as pltpu
from jax.experimental.pallas import tpu_sc as plsc

B, N, D, H, L, TOPK = 16, 4096, 768, 12, 2, 16
HID = D // H
DH = D // H
S = TOPK + 1
SCORE_ROWS = 4096

_HI = lax.Precision.HIGHEST


def _dot_t(a, w):
    return lax.dot_general(a, w, (((1,), (1,)), ((), ())),
                           precision=_HI, preferred_element_type=jnp.float32)


def _ln(x, g, b):
    m = jnp.mean(x, axis=-1, keepdims=True)
    xc = x - m
    v = jnp.mean(xc * xc, axis=-1, keepdims=True)
    return xc / jnp.sqrt(v + 1e-5) * g + b


def _score_body(x_ref, g_ref, b_ref, w1_ref, b1_ref, w2_ref, b2_ref, out_ref):
    xb = x_ref[...]
    ln = _ln(xb, g_ref[...], b_ref[...])
    h = lax.dot_general(ln.astype(jnp.bfloat16), w1_ref[...].astype(jnp.bfloat16),
                        (((1,), (1,)), ((), ())),
                        preferred_element_type=jnp.float32) + b1_ref[...]
    h = h * 0.5 * (1.0 + lax.erf(h * (1.0 / jnp.sqrt(2.0)).astype(jnp.float32)))
    hb = h.astype(jnp.bfloat16).astype(jnp.float32)
    wb = w2_ref[...].astype(jnp.bfloat16).astype(jnp.float32)
    s = jnp.sum(hb * wb, axis=1, keepdims=True) + b2_ref[...]
    out_ref[...] = jax.nn.sigmoid(s)


def _score(x2d, g, b, w1, b1, w2row, b2):
    bn = x2d.shape[0]
    return pl.pallas_call(
        _score_body,
        grid=(bn // SCORE_ROWS,),
        in_specs=[
            pl.BlockSpec((SCORE_ROWS, D), lambda i: (i, 0)),
            pl.BlockSpec((D,), lambda i: (0,)),
            pl.BlockSpec((D,), lambda i: (0,)),
            pl.BlockSpec((HID, D), lambda i: (0, 0)),
            pl.BlockSpec((HID,), lambda i: (0,)),
            pl.BlockSpec((HID,), lambda i: (0,)),
            pl.BlockSpec((1,), lambda i: (0,)),
        ],
        out_specs=pl.BlockSpec((SCORE_ROWS, 1), lambda i: (i, 0)),
        out_shape=jax.ShapeDtypeStruct((bn, 1), jnp.float32),
    )(x2d, g, b, w1, b1, w2row, b2)


def _topk_gather(w, xflat):
    mesh = plsc.VectorSubcoreMesh(core_axis_name="c", subcore_axis_name="s")

    @functools.partial(
        pl.kernel,
        mesh=mesh,
        compiler_params=pltpu.CompilerParams(needs_layout_passes=False),
        out_type=(jax.ShapeDtypeStruct((B * TOPK, D), jnp.float32),
                  jax.ShapeDtypeStruct((B * TOPK,), jnp.float32)),
        scratch_types=[
            pltpu.VMEM((N,), jnp.float32),
            pltpu.VMEM((TOPK,), jnp.int32),
            pltpu.VMEM((TOPK,), jnp.float32),
            pltpu.VMEM((TOPK, D), jnp.float32),
            pltpu.SemaphoreType.DMA,
        ],
    )
    def k(w_hbm, x_hbm, rows_out, vals_out, w_v, idx_v, val_v, rows_v, sem):
        wid = lax.axis_index("s") * 2 + lax.axis_index("c")

        @pl.when(wid < B)
        def _():
            bag = wid
            pltpu.sync_copy(w_hbm.at[bag], w_v)
            lanes = lax.iota(jnp.int32, 16)
            cur_v, cur_i = plsc.sort_key_val(w_v[pl.ds(0, 16)], lanes)

            def body(c, carry):
                cv, ci = carry
                sv, si = plsc.sort_key_val(w_v[pl.ds(c * 16, 16)],
                                           lanes + c * 16)
                rv = lax.rev(sv, (0,))
                ri = lax.rev(si, (0,))
                keep = cv >= rv
                nv, ni = plsc.sort_key_val(jnp.where(keep, cv, rv),
                                           jnp.where(keep, ci, ri))
                return (nv, ni)

            cur_v, cur_i = lax.fori_loop(1, N // 16, body, (cur_v, cur_i))
            idx_v[...] = cur_i + bag * N
            val_v[...] = cur_v
            pltpu.async_copy(x_hbm.at[idx_v], rows_v, sem).wait()
            pltpu.sync_copy(rows_v, rows_out.at[pl.ds(bag * TOPK, TOPK)])
            pltpu.sync_copy(val_v, vals_out.at[pl.ds(bag * TOPK, TOPK)])

    return k(w, xflat)


_T = B * S


def _bdot(a, w):
    return lax.dot_general(a.astype(jnp.bfloat16), w.astype(jnp.bfloat16),
                           (((1,), (1,)), ((), ())),
                           preferred_element_type=jnp.float32)


def _attn_body(emb_ref, tw_ref, cls_ref, g_ref, b_ref, inw_ref, inb_ref,
               outw_ref, outb_ref, o_ref, z_scr, a_scr):
    g = g_ref[...]
    bb = b_ref[...]
    scale = (1.0 / jnp.sqrt(float(DH))).astype(jnp.float32)

    def layer(a, cls_only):
        qkv = _bdot(a, inw_ref[0]) + inb_ref[0]
        q = qkv[:, :D]
        kk = qkv[:, D:2 * D]
        vv = qkv[:, 2 * D:]
        if cls_only:
            q = q.reshape(B, S, D)[:, 0, :]
            m = lax.broadcasted_iota(jnp.int32, (B, _T), 0) == \
                lax.broadcasted_iota(jnp.int32, (B, _T), 1) // S
        else:
            m = lax.broadcasted_iota(jnp.int32, (_T, _T), 0) // S == \
                lax.broadcasted_iota(jnp.int32, (_T, _T), 1) // S
        outs = []
        for hh in range(H):
            lo = hh * DH
            sc = _bdot(q[:, lo:lo + DH], kk[:, lo:lo + DH]) * scale
            sc = jnp.where(m, sc, -1e30)
            sc = sc - jnp.max(sc, axis=1, keepdims=True)
            e = jnp.exp(sc)
            p = e / jnp.sum(e, axis=1, keepdims=True)
            outs.append(lax.dot_general(
                p.astype(jnp.bfloat16), vv[:, lo:lo + DH].astype(jnp.bfloat16),
                (((1,), (0,)), ((), ())),
                preferred_element_type=jnp.float32))
        o = jnp.concatenate(outs, axis=1)
        return _bdot(o, outw_ref[0]) + outb_ref[0]

    @pl.when(pl.program_id(0) == 0)
    def _():
        ew = (emb_ref[...] * tw_ref[...]).reshape(B, TOPK, D)
        cls = jnp.broadcast_to(cls_ref[...].reshape(1, 1, D), (B, 1, D))
        zc = jnp.concatenate([cls, ew], axis=1).reshape(_T, D)
        z = _ln(zc, g, bb)
        z_scr[...] = z
        a_scr[...] = _ln(z + layer(z, False), g, bb)

    @pl.when(pl.program_id(0) == 1)
    def _():
        o2 = layer(a_scr[...], True)
        z_cls = z_scr[...].reshape(B, S, D)[:, 0, :]
        o_ref[...] = _ln(z_cls + o2, g, bb)


def _attn(emb, tw, cls, g, b, in_w, in_b, out_w, out_b):
    return pl.pallas_call(
        _attn_body,
        grid=(L,),
        in_specs=[
            pl.BlockSpec((B * TOPK, D), lambda l: (0, 0)),
            pl.BlockSpec((B * TOPK, 1), lambda l: (0, 0)),
            pl.BlockSpec((1, D), lambda l: (0, 0)),
            pl.BlockSpec((D,), lambda l: (0,)),
            pl.BlockSpec((D,), lambda l: (0,)),
            pl.BlockSpec((1, 3 * D, D), lambda l: (l, 0, 0)),
            pl.BlockSpec((1, 1, 3 * D), lambda l: (l, 0, 0)),
            pl.BlockSpec((1, D, D), lambda l: (l, 0, 0)),
            pl.BlockSpec((1, 1, D), lambda l: (l, 0, 0)),
        ],
        out_specs=pl.BlockSpec((B, D), lambda l: (0, 0)),
        out_shape=jax.ShapeDtypeStruct((B, D), jnp.float32),
        scratch_shapes=[pltpu.VMEM((_T, D), jnp.float32),
                        pltpu.VMEM((_T, D), jnp.float32)],
    )(emb, tw, cls, g, b, in_w, in_b.reshape(L, 1, 3 * D), out_w,
      out_b.reshape(L, 1, D))


def kernel(x, cls_token, ln_g, ln_b, aw_ln_g, aw_ln_b, aw_w1, aw_b1, aw_w2,
           aw_b2, in_w, in_b, out_w, out_b):
    x2d = x.reshape(B * N, D)
    w = _score(x2d, aw_ln_g, aw_ln_b, aw_w1, aw_b1, aw_w2.reshape(HID), aw_b2)
    rows, vals = _topk_gather(w.reshape(B, N), x2d)
    return _attn(rows, vals.reshape(B * TOPK, 1),
                 cls_token.reshape(1, D), ln_g, ln_b, in_w, in_b, out_w, out_b)
